# baseline (speedup 1.0000x reference)
import jax
import jax.numpy as jnp
import numpy as np
from jax.experimental import pallas as pl
from jax.experimental.pallas import tpu as pltpu

_PI = float(np.arccos(0.0) * 2.0)
_D225 = _PI / 8
_D675 = 3 * _PI / 8
_D1125 = 5 * _PI / 8
_D1575 = 7 * _PI / 8
_D180 = _PI

_W = 512


def _select_nmax(tv, pair_h, pair_d45, pair_v, pair_d135):
    hm = (tv < _D225) | (tv >= _D1575)
    return jnp.where(
        hm, pair_h,
        jnp.where(tv < _D675, pair_d45,
                  jnp.where(tv < _D1125, pair_v, pair_d135)))


_IMGS = 4


def _nms_image(g, t):
    tv = jnp.abs(t)

    left = jnp.concatenate([g[:, :, :1], g[:, :, :-1]], axis=2)
    right = jnp.concatenate([g[:, :, 1:], g[:, :, -1:]], axis=2)

    def up(x):
        return jnp.concatenate([x[:, :1], x[:, :-1]], axis=1)

    def down(x):
        return jnp.concatenate([x[:, 1:], x[:, -1:]], axis=1)

    pair_h = jnp.maximum(left, right)
    pair_d45 = jnp.maximum(up(right), down(left))
    pair_v = jnp.maximum(up(g), down(g))
    pair_d135 = jnp.maximum(up(left), down(right))

    nmax = _select_nmax(tv, pair_h, pair_d45, pair_v, pair_d135)
    keep = (g >= nmax) & (tv <= _D180)
    return jnp.where(keep, g, jnp.zeros_like(g))


def _tc_body(g_ref, t_ref, o_ref):
    o_ref[...] = _nms_image(g_ref[...], t_ref[...])


def _tc_nms(g3, t3):
    B = g3.shape[0]
    return pl.pallas_call(
        _tc_body,
        grid=(B // _IMGS,),
        in_specs=[
            pl.BlockSpec((_IMGS, _W, _W), lambda i: (i, 0, 0)),
            pl.BlockSpec((_IMGS, _W, _W), lambda i: (i, 0, 0)),
        ],
        out_specs=pl.BlockSpec((_IMGS, _W, _W), lambda i: (i, 0, 0)),
        out_shape=jax.ShapeDtypeStruct((B, _W, _W), jnp.float32),
        compiler_params=pltpu.CompilerParams(
            dimension_semantics=("parallel",)),
    )(g3, t3)


def kernel(g, t):
    B, _, H, W = g.shape
    out = _tc_nms(g.reshape(B, H, W), t.reshape(B, H, W))
    return out.reshape(B, 1, H, W)
